# baseline (speedup 1.0000x reference)
import functools

import jax
import jax.numpy as jnp
from jax import lax
from jax.experimental import pallas as pl
from jax.experimental.pallas import tpu as pltpu
from jax.experimental.pallas import tpu_sc as plsc

NC = 2
NS = 16
NW = NC * NS
ECH = 128
DCH = 128


def _mesh():
    return plsc.VectorSubcoreMesh(
        core_axis_name="c", subcore_axis_name="s", num_cores=NC, num_subcores=NS)


def _deg_kernel(n_acc, per_w):
    @functools.partial(
        pl.kernel,
        mesh=_mesh(),
        out_type=jax.ShapeDtypeStruct((NW, n_acc), jnp.float32),
        scratch_types=[
            pltpu.VMEM((per_w,), jnp.int32),
            pltpu.VMEM((n_acc,), jnp.float32),
        ],
        compiler_params=pltpu.CompilerParams(needs_layout_passes=False),
    )
    def k(dst_hbm, zeros_hbm, out_hbm, dst_v, ldeg):
        c = lax.axis_index("c")
        s = lax.axis_index("s")
        w = c * NS + s
        pltpu.sync_copy(zeros_hbm, ldeg)
        pltpu.sync_copy(dst_hbm.at[w], dst_v)
        ones16 = jnp.ones((16,), jnp.float32)

        def body(i, carry):
            idx = dst_v[pl.ds(i * 16, 16)]
            plsc.addupdate_scatter(ldeg, [idx], ones16)
            return carry

        lax.fori_loop(0, per_w // 16, body, 0)
        pltpu.sync_copy(ldeg, out_hbm.at[w])

    return k


def _edge_kernel(n_acc, d, n_chunks):
    init_rows = n_acc // NS
    assert n_chunks % 4 == 0 and n_chunks >= 4

    @functools.partial(
        pl.kernel,
        mesh=_mesh(),
        out_type=jax.ShapeDtypeStruct((NC, n_acc, d), jnp.float32),
        scratch_types=[
            pltpu.VMEM_SHARED((n_acc, d), jnp.float32),
            pltpu.VMEM((4, ECH), jnp.int32),
            pltpu.VMEM((4, ECH), jnp.int32),
            pltpu.VMEM((ECH, d), jnp.float32),
            pltpu.VMEM((ECH, d), jnp.float32),
            pltpu.SemaphoreType.DMA,
            pltpu.SemaphoreType.DMA,
            pltpu.SemaphoreType.DMA,
            pltpu.SemaphoreType.DMA,
            pltpu.SemaphoreType.DMA,
            pltpu.SemaphoreType.DMA,
        ],
    )
    def k(y_hbm, src_hbm, dst_hbm, zeros_hbm, out_hbm, acc, src_r, dst_r,
          rows0, rows1, si0, si1, si2, si3, sr0, sr1):
        c = lax.axis_index("c")
        s = lax.axis_index("s")
        w = c * NS + s
        pltpu.sync_copy(zeros_hbm.at[pl.ds(s * init_rows, init_rows)],
                        acc.at[pl.ds(s * init_rows, init_rows)])
        plsc.subcore_barrier()

        rows = (rows0, rows1)
        rsem = (sr0, sr1)
        isem = (si0, si1, si2, si3)

        def fetch_idx(t, slot):
            pltpu.async_copy(src_hbm.at[w, t], src_r.at[slot], isem[slot])
            pltpu.async_copy(dst_hbm.at[w, t], dst_r.at[slot], isem[slot])

        def wait_idx(slot):
            pltpu.make_async_copy(
                src_hbm.at[w, 0], src_r.at[slot], isem[slot]).wait()
            pltpu.make_async_copy(
                dst_hbm.at[w, 0], dst_r.at[slot], isem[slot]).wait()

        for t in range(4):
            fetch_idx(t, t)
        for b in range(2):
            wait_idx(b)
            pltpu.async_copy(y_hbm.at[src_r.at[b]], rows[b], rsem[b])

        def body(j, carry):
            for b in range(4):
                i = j * 4 + b
                pltpu.make_async_copy(
                    y_hbm.at[src_r.at[b]], rows[b % 2], rsem[b % 2]).wait()
                pltpu.sync_copy(rows[b % 2], acc.at[dst_r.at[b]], add=True)
                fetch_idx(jnp.minimum(i + 4, n_chunks - 1), b)
                wait_idx((b + 2) % 4)
                pltpu.async_copy(
                    y_hbm.at[src_r.at[(b + 2) % 4]], rows[b % 2],
                    rsem[b % 2])
            return carry

        lax.fori_loop(0, n_chunks // 4, body, 0)
        for b in range(2):
            pltpu.make_async_copy(
                y_hbm.at[src_r.at[b]], rows[b], rsem[b]).wait()
        wait_idx(2)
        wait_idx(3)
        plsc.subcore_barrier()
        pltpu.sync_copy(acc.at[pl.ds(s * init_rows, init_rows)],
                        out_hbm.at[c, pl.ds(s * init_rows, init_rows)])

    return k


def _decode_gather_kernel(n, d, n_chunks):
    per_w = n_chunks * DCH

    @functools.partial(
        pl.kernel,
        mesh=_mesh(),
        out_type=(
            jax.ShapeDtypeStruct((NW * per_w, d), jnp.float32),
            jax.ShapeDtypeStruct((NW * per_w, d), jnp.float32),
        ),
        scratch_types=[
            pltpu.VMEM((n_chunks, DCH), jnp.int32),
            pltpu.VMEM((n_chunks, DCH), jnp.int32),
            pltpu.VMEM((DCH, d), jnp.float32),
            pltpu.VMEM((DCH, d), jnp.float32),
            pltpu.VMEM((DCH, d), jnp.float32),
            pltpu.VMEM((DCH, d), jnp.float32),
            pltpu.SemaphoreType.DMA,
            pltpu.SemaphoreType.DMA,
            pltpu.SemaphoreType.DMA,
            pltpu.SemaphoreType.DMA,
        ],
    )
    def k(z_hbm, i0_hbm, i1_hbm, a_hbm, b_hbm, i0_v, i1_v, ra0, ra1, rb0, rb1,
          sa0, sa1, sb0, sb1):
        c = lax.axis_index("c")
        s = lax.axis_index("s")
        w = c * NS + s
        base = w * per_w
        pltpu.sync_copy(i0_hbm.at[w], i0_v)
        pltpu.sync_copy(i1_hbm.at[w], i1_v)

        ra = (ra0, ra1)
        rb = (rb0, rb1)
        sa = (sa0, sa1)
        sb = (sb0, sb1)
        for b in range(2):
            pltpu.async_copy(z_hbm.at[i0_v.at[b]], ra[b], sa[b])
            pltpu.async_copy(z_hbm.at[i1_v.at[b]], rb[b], sb[b])

        def body(j, carry):
            for b in range(2):
                i = j * 2 + b
                nxt = jnp.minimum(i + 2, n_chunks - 1)
                pltpu.make_async_copy(
                    z_hbm.at[i0_v.at[i]], ra[b], sa[b]).wait()
                pltpu.sync_copy(ra[b], a_hbm.at[pl.ds(base + i * DCH, DCH)])
                pltpu.async_copy(z_hbm.at[i0_v.at[nxt]], ra[b], sa[b])
                pltpu.make_async_copy(
                    z_hbm.at[i1_v.at[i]], rb[b], sb[b]).wait()
                pltpu.sync_copy(rb[b], b_hbm.at[pl.ds(base + i * DCH, DCH)])
                pltpu.async_copy(z_hbm.at[i1_v.at[nxt]], rb[b], sb[b])
            return carry

        lax.fori_loop(0, n_chunks // 2, body, 0)
        for b in range(2):
            pltpu.make_async_copy(z_hbm.at[i0_v.at[0]], ra[b], sa[b]).wait()
            pltpu.make_async_copy(z_hbm.at[i1_v.at[0]], rb[b], sb[b]).wait()

    return k


def _tc_dinv_body(degp_ref, dinv_ref):
    deg = jnp.sum(degp_ref[...], axis=0) + 1.0
    dinv_ref[...] = lax.rsqrt(deg)[:, None]


def _tc_y1_body(x_ref, w_ref, dinv_ref, y_ref):
    xw = jnp.dot(x_ref[...], w_ref[...], preferred_element_type=jnp.float32)
    y_ref[...] = xw * dinv_ref[...]


def _tc_mid_body(p_ref, y1_ref, dinv_ref, b1_ref, zs_ref):
    dinv = dinv_ref[...]
    h = (p_ref[0] + p_ref[1] + y1_ref[...]) * dinv + b1_ref[...]
    zs_ref[...] = jnp.maximum(h, 0.0) * dinv


def _tc_z2_body(p_ref, zs_ref, dinv_ref, b2_ref, w2_ref, z2_ref):
    u = (p_ref[0] + p_ref[1] + zs_ref[...]) * dinv_ref[...]
    z2 = jnp.dot(u, w2_ref[...], preferred_element_type=jnp.float32) + b2_ref[...]
    d_out = z2.shape[1]
    z2_ref[:, :d_out] = z2
    z2_ref[:, d_out:] = jnp.zeros_like(z2_ref[:, d_out:])


def _tc_dot_body(a_ref, b_ref, o_ref):
    o_ref[...] = jnp.sum(a_ref[...] * b_ref[...], axis=-1, keepdims=True)


def _row_blocks(n):
    for bs in (1000, 500, 250, 125, 100, 50, 25, 8, 5, 4, 2, 1):
        if n % bs == 0:
            return bs
    return n


def _pad_chunks(a, fill, per_w_chunks, ch):
    total = NW * per_w_chunks * ch
    pad = total - a.shape[0]
    a = jnp.concatenate([a.astype(jnp.int32),
                         jnp.full((pad,), fill, dtype=jnp.int32)])
    return a.reshape(NW, per_w_chunks, ch)


def kernel(x, edge_index, edge_label_index, W1, b1, W2, b2):
    n, d_in = x.shape
    d_hid = W1.shape[1]
    d_out = W2.shape[1]
    e = edge_index.shape[1]
    e_lab = edge_label_index.shape[1]

    n_acc = ((n + 1 + NS * 8 - 1) // (NS * 8)) * (NS * 8)
    dump = n

    ec = -(-e // (NW * ECH))
    lc = -(-e_lab // (NW * DCH))
    ec = ((ec + 3) // 4) * 4
    lc += lc % 2
    src3 = _pad_chunks(edge_index[0], 0, ec, ECH)
    dst3 = _pad_chunks(edge_index[1], dump, ec, ECH)
    li0 = _pad_chunks(edge_label_index[0], 0, lc, DCH)
    li1 = _pad_chunks(edge_label_index[1], 0, lc, DCH)

    zeros_deg = jnp.zeros((n_acc,), jnp.float32)
    zeros1 = jnp.zeros((n_acc, d_hid), jnp.float32)

    deg_p = _deg_kernel(n_acc, ec * ECH)(dst3.reshape(NW, ec * ECH), zeros_deg)

    dinv = pl.pallas_call(
        _tc_dinv_body,
        grid=(1,),
        in_specs=[pl.BlockSpec((NW, n_acc), lambda i: (0, 0))],
        out_specs=pl.BlockSpec((n_acc, 1), lambda i: (0, 0)),
        out_shape=jax.ShapeDtypeStruct((n_acc, 1), jnp.float32),
    )(deg_p)

    bs = _row_blocks(n)
    grid = (n // bs,)
    deg_spec = pl.BlockSpec((bs, 1), lambda i: (i, 0))
    row_spec = lambda d: pl.BlockSpec((bs, d), lambda i: (i, 0))
    mat_spec = lambda a, b: pl.BlockSpec((a, b), lambda i: (0, 0))
    p_spec = lambda d: pl.BlockSpec((NC, bs, d), lambda i: (0, i, 0))

    y1 = pl.pallas_call(
        _tc_y1_body,
        grid=grid,
        in_specs=[row_spec(d_in), mat_spec(d_in, d_hid), deg_spec],
        out_specs=row_spec(d_hid),
        out_shape=jax.ShapeDtypeStruct((n, d_hid), jnp.float32),
    )(x, W1, dinv)

    p1 = _edge_kernel(n_acc, d_hid, ec)(y1, src3, dst3, zeros1)

    zs = pl.pallas_call(
        _tc_mid_body,
        grid=grid,
        in_specs=[p_spec(d_hid), row_spec(d_hid), deg_spec,
                  mat_spec(1, d_hid)],
        out_specs=row_spec(d_hid),
        out_shape=jax.ShapeDtypeStruct((n, d_hid), jnp.float32),
    )(p1, y1, dinv, b1.reshape(1, d_hid))

    p2 = _edge_kernel(n_acc, d_hid, ec)(zs, src3, dst3, zeros1)

    z2 = pl.pallas_call(
        _tc_z2_body,
        grid=grid,
        in_specs=[p_spec(d_hid), row_spec(d_hid), deg_spec,
                  mat_spec(1, d_out), mat_spec(d_hid, d_out)],
        out_specs=row_spec(d_hid),
        out_shape=jax.ShapeDtypeStruct((n, d_hid), jnp.float32),
    )(p2, zs, dinv, b2.reshape(1, d_out), W2)

    a_rows, b_rows = _decode_gather_kernel(n, d_hid, lc)(z2, li0, li1)

    e_pad = NW * lc * DCH
    dbs = min(2048, e_pad)
    while e_pad % dbs:
        dbs //= 2
    logits2 = pl.pallas_call(
        _tc_dot_body,
        grid=(e_pad // dbs,),
        in_specs=[pl.BlockSpec((dbs, d_hid), lambda i: (i, 0)),
                  pl.BlockSpec((dbs, d_hid), lambda i: (i, 0))],
        out_specs=pl.BlockSpec((dbs, 1), lambda i: (i, 0)),
        out_shape=jax.ShapeDtypeStruct((e_pad, 1), jnp.float32),
    )(a_rows, b_rows)

    return logits2[:e_lab, 0]


# speedup vs baseline: 1.7866x; 1.7866x over previous
import functools

import jax
import jax.numpy as jnp
from jax import lax
from jax.experimental import pallas as pl
from jax.experimental.pallas import tpu as pltpu
from jax.experimental.pallas import tpu_sc as plsc

NC = 2
NS = 16
NW = NC * NS
ECH = 128
DCH = 128


def _mesh():
    return plsc.VectorSubcoreMesh(
        core_axis_name="c", subcore_axis_name="s", num_cores=NC, num_subcores=NS)


def _deg_kernel(n_acc, per_w):
    @functools.partial(
        pl.kernel,
        mesh=_mesh(),
        out_type=jax.ShapeDtypeStruct((NW, n_acc), jnp.float32),
        scratch_types=[
            pltpu.VMEM((per_w,), jnp.int32),
            pltpu.VMEM((n_acc,), jnp.float32),
        ],
        compiler_params=pltpu.CompilerParams(needs_layout_passes=False),
    )
    def k(dst_hbm, zeros_hbm, out_hbm, dst_v, ldeg):
        c = lax.axis_index("c")
        s = lax.axis_index("s")
        w = c * NS + s
        pltpu.sync_copy(zeros_hbm, ldeg)
        pltpu.sync_copy(dst_hbm.at[w], dst_v)
        ones16 = jnp.ones((16,), jnp.float32)

        def body(i, carry):
            idx = dst_v[pl.ds(i * 16, 16)]
            plsc.addupdate_scatter(ldeg, [idx], ones16)
            return carry

        lax.fori_loop(0, per_w // 16, body, 0)
        pltpu.sync_copy(ldeg, out_hbm.at[w])

    return k


def _edge_kernel(n_acc, d, n_chunks):
    init_rows = n_acc // NS

    @functools.partial(
        pl.kernel,
        mesh=_mesh(),
        out_type=jax.ShapeDtypeStruct((NC, n_acc, d), jnp.float32),
        scratch_types=[
            pltpu.VMEM_SHARED((n_acc, d), jnp.float32),
            pltpu.VMEM((n_chunks, ECH), jnp.int32),
            pltpu.VMEM((n_chunks, ECH), jnp.int32),
            pltpu.VMEM((ECH, d), jnp.float32),
            pltpu.SemaphoreType.DMA,
        ],
    )
    def k(y_hbm, src_hbm, dst_hbm, zeros_hbm, out_hbm, acc, src_v, dst_v,
          rows_v, sem):
        c = lax.axis_index("c")
        s = lax.axis_index("s")
        w = c * NS + s
        pltpu.sync_copy(zeros_hbm.at[pl.ds(s * init_rows, init_rows)],
                        acc.at[pl.ds(s * init_rows, init_rows)])
        pltpu.sync_copy(src_hbm.at[w], src_v)
        pltpu.sync_copy(dst_hbm.at[w], dst_v)
        plsc.subcore_barrier()

        def body(j, carry):
            pltpu.async_copy(y_hbm.at[src_v.at[j]], rows_v, sem).wait()
            pltpu.sync_copy(rows_v, acc.at[dst_v.at[j]], add=True)
            return carry

        lax.fori_loop(0, n_chunks, body, 0)
        plsc.subcore_barrier()
        pltpu.sync_copy(acc.at[pl.ds(s * init_rows, init_rows)],
                        out_hbm.at[c, pl.ds(s * init_rows, init_rows)])

    return k


def _decode_gather_kernel(n, d, n_chunks):
    per_w = n_chunks * DCH

    @functools.partial(
        pl.kernel,
        mesh=_mesh(),
        out_type=(
            jax.ShapeDtypeStruct((NW * per_w, d), jnp.float32),
            jax.ShapeDtypeStruct((NW * per_w, d), jnp.float32),
        ),
        scratch_types=[
            pltpu.VMEM((n_chunks, DCH), jnp.int32),
            pltpu.VMEM((n_chunks, DCH), jnp.int32),
            pltpu.VMEM((DCH, d), jnp.float32),
            pltpu.VMEM((DCH, d), jnp.float32),
            pltpu.SemaphoreType.DMA,
            pltpu.SemaphoreType.DMA,
        ],
    )
    def k(z_hbm, i0_hbm, i1_hbm, a_hbm, b_hbm, i0_v, i1_v, ra_v, rb_v, s0, s1):
        c = lax.axis_index("c")
        s = lax.axis_index("s")
        w = c * NS + s
        base = w * per_w
        pltpu.sync_copy(i0_hbm.at[w], i0_v)
        pltpu.sync_copy(i1_hbm.at[w], i1_v)

        def body(j, carry):
            cp0 = pltpu.async_copy(z_hbm.at[i0_v.at[j]], ra_v, s0)
            cp1 = pltpu.async_copy(z_hbm.at[i1_v.at[j]], rb_v, s1)
            cp0.wait()
            cp1.wait()
            pltpu.sync_copy(ra_v, a_hbm.at[pl.ds(base + j * DCH, DCH)])
            pltpu.sync_copy(rb_v, b_hbm.at[pl.ds(base + j * DCH, DCH)])
            return carry

        lax.fori_loop(0, n_chunks, body, 0)

    return k


def _tc_dinv_body(degp_ref, dinv_ref):
    deg = jnp.sum(degp_ref[...], axis=0) + 1.0
    dinv_ref[...] = lax.rsqrt(deg)[:, None]


def _tc_y1_body(x_ref, w_ref, dinv_ref, y_ref):
    xw = jnp.dot(x_ref[...], w_ref[...], preferred_element_type=jnp.float32)
    y_ref[...] = xw * dinv_ref[...]


def _tc_mid_body(p_ref, y1_ref, dinv_ref, b1_ref, zs_ref):
    dinv = dinv_ref[...]
    h = (p_ref[0] + p_ref[1] + y1_ref[...]) * dinv + b1_ref[...]
    zs_ref[...] = jnp.maximum(h, 0.0) * dinv


def _tc_z2_body(p_ref, zs_ref, dinv_ref, b2_ref, w2_ref, z2_ref):
    u = (p_ref[0] + p_ref[1] + zs_ref[...]) * dinv_ref[...]
    z2 = jnp.dot(u, w2_ref[...], preferred_element_type=jnp.float32) + b2_ref[...]
    d_out = z2.shape[1]
    z2_ref[:, :d_out] = z2
    z2_ref[:, d_out:] = jnp.zeros_like(z2_ref[:, d_out:])


def _tc_dot_body(a_ref, b_ref, o_ref):
    o_ref[...] = jnp.sum(a_ref[...] * b_ref[...], axis=-1, keepdims=True)


def _row_blocks(n):
    for bs in (1000, 500, 250, 125, 100, 50, 25, 8, 5, 4, 2, 1):
        if n % bs == 0:
            return bs
    return n


def _pad_chunks(a, fill, per_w_chunks, ch):
    total = NW * per_w_chunks * ch
    pad = total - a.shape[0]
    a = jnp.concatenate([a.astype(jnp.int32),
                         jnp.full((pad,), fill, dtype=jnp.int32)])
    return a.reshape(NW, per_w_chunks, ch)


def kernel(x, edge_index, edge_label_index, W1, b1, W2, b2):
    n, d_in = x.shape
    d_hid = W1.shape[1]
    d_out = W2.shape[1]
    e = edge_index.shape[1]
    e_lab = edge_label_index.shape[1]

    n_acc = ((n + 1 + NS * 8 - 1) // (NS * 8)) * (NS * 8)
    dump = n

    ec = -(-e // (NW * ECH))
    lc = -(-e_lab // (NW * DCH))

    src3 = _pad_chunks(edge_index[0], 0, ec, ECH)
    dst3 = _pad_chunks(edge_index[1], dump, ec, ECH)
    li0 = _pad_chunks(edge_label_index[0], 0, lc, DCH)
    li1 = _pad_chunks(edge_label_index[1], 0, lc, DCH)

    zeros_deg = jnp.zeros((n_acc,), jnp.float32)
    zeros1 = jnp.zeros((n_acc, d_hid), jnp.float32)

    deg_p = _deg_kernel(n_acc, ec * ECH)(dst3.reshape(NW, ec * ECH), zeros_deg)

    dinv = pl.pallas_call(
        _tc_dinv_body,
        grid=(1,),
        in_specs=[pl.BlockSpec((NW, n_acc), lambda i: (0, 0))],
        out_specs=pl.BlockSpec((n_acc, 1), lambda i: (0, 0)),
        out_shape=jax.ShapeDtypeStruct((n_acc, 1), jnp.float32),
    )(deg_p)

    bs = _row_blocks(n)
    grid = (n // bs,)
    deg_spec = pl.BlockSpec((bs, 1), lambda i: (i, 0))
    row_spec = lambda d: pl.BlockSpec((bs, d), lambda i: (i, 0))
    mat_spec = lambda a, b: pl.BlockSpec((a, b), lambda i: (0, 0))
    p_spec = lambda d: pl.BlockSpec((NC, bs, d), lambda i: (0, i, 0))

    y1 = pl.pallas_call(
        _tc_y1_body,
        grid=grid,
        in_specs=[row_spec(d_in), mat_spec(d_in, d_hid), deg_spec],
        out_specs=row_spec(d_hid),
        out_shape=jax.ShapeDtypeStruct((n, d_hid), jnp.float32),
    )(x, W1, dinv)

    p1 = _edge_kernel(n_acc, d_hid, ec)(y1, src3, dst3, zeros1)

    zs = pl.pallas_call(
        _tc_mid_body,
        grid=grid,
        in_specs=[p_spec(d_hid), row_spec(d_hid), deg_spec,
                  mat_spec(1, d_hid)],
        out_specs=row_spec(d_hid),
        out_shape=jax.ShapeDtypeStruct((n, d_hid), jnp.float32),
    )(p1, y1, dinv, b1.reshape(1, d_hid))

    p2 = _edge_kernel(n_acc, d_hid, ec)(zs, src3, dst3, zeros1)

    z2 = pl.pallas_call(
        _tc_z2_body,
        grid=grid,
        in_specs=[p_spec(d_hid), row_spec(d_hid), deg_spec,
                  mat_spec(1, d_out), mat_spec(d_hid, d_out)],
        out_specs=row_spec(d_hid),
        out_shape=jax.ShapeDtypeStruct((n, d_hid), jnp.float32),
    )(p2, zs, dinv, b2.reshape(1, d_out), W2)

    a_rows, b_rows = _decode_gather_kernel(n, d_hid, lc)(z2, li0, li1)

    e_pad = NW * lc * DCH
    dbs = min(2048, e_pad)
    while e_pad % dbs:
        dbs //= 2
    logits2 = pl.pallas_call(
        _tc_dot_body,
        grid=(e_pad // dbs,),
        in_specs=[pl.BlockSpec((dbs, d_hid), lambda i: (i, 0)),
                  pl.BlockSpec((dbs, d_hid), lambda i: (i, 0))],
        out_specs=pl.BlockSpec((dbs, 1), lambda i: (i, 0)),
        out_shape=jax.ShapeDtypeStruct((e_pad, 1), jnp.float32),
    )(a_rows, b_rows)

    return logits2[:e_lab, 0]


# speedup vs baseline: 2.0572x; 1.1514x over previous
import functools

import jax
import jax.numpy as jnp
from jax import lax
from jax.experimental import pallas as pl
from jax.experimental.pallas import tpu as pltpu
from jax.experimental.pallas import tpu_sc as plsc

NC = 2
NS = 16
NW = NC * NS
ECH = 128
DCH = 128


def _mesh():
    return plsc.VectorSubcoreMesh(
        core_axis_name="c", subcore_axis_name="s", num_cores=NC, num_subcores=NS)


def _deg_kernel(n_acc, per_w):
    @functools.partial(
        pl.kernel,
        mesh=_mesh(),
        out_type=jax.ShapeDtypeStruct((NW, n_acc), jnp.float32),
        scratch_types=[
            pltpu.VMEM((per_w,), jnp.int32),
            pltpu.VMEM((n_acc,), jnp.float32),
        ],
        compiler_params=pltpu.CompilerParams(needs_layout_passes=False),
    )
    def k(dst_hbm, zeros_hbm, out_hbm, dst_v, ldeg):
        c = lax.axis_index("c")
        s = lax.axis_index("s")
        w = c * NS + s
        pltpu.sync_copy(zeros_hbm, ldeg)
        pltpu.sync_copy(dst_hbm.at[w], dst_v)
        ones16 = jnp.ones((16,), jnp.float32)

        def body(i, carry):
            idx = dst_v[pl.ds(i * 16, 16)]
            plsc.addupdate_scatter(ldeg, [idx], ones16)
            return carry

        lax.fori_loop(0, per_w // 16, body, 0)
        pltpu.sync_copy(ldeg, out_hbm.at[w])

    return k


def _edge_kernel(n_acc, d, n_chunks, nc0, nc1):
    init_rows = n_acc // NS

    @functools.partial(
        pl.kernel,
        mesh=_mesh(),
        out_type=jax.ShapeDtypeStruct((NC, n_acc, d), jnp.float32),
        scratch_types=[
            pltpu.VMEM_SHARED((n_acc, d), jnp.float32),
            pltpu.VMEM((n_chunks, ECH), jnp.int32),
            pltpu.VMEM((n_chunks, ECH), jnp.int32),
            pltpu.VMEM((ECH, d), jnp.float32),
            pltpu.SemaphoreType.DMA,
        ],
    )
    def k(y_hbm, src_hbm, dst_hbm, zeros_hbm, out_hbm, acc, src_v, dst_v,
          rows_v, sem):
        c = lax.axis_index("c")
        s = lax.axis_index("s")
        w = c * NS + s
        pltpu.sync_copy(zeros_hbm.at[pl.ds(s * init_rows, init_rows)],
                        acc.at[pl.ds(s * init_rows, init_rows)])
        pltpu.sync_copy(src_hbm.at[w], src_v)
        pltpu.sync_copy(dst_hbm.at[w], dst_v)
        plsc.subcore_barrier()

        nb = jnp.where(c == 0, nc0, nc1)

        def body(j, carry):
            @pl.when(j < nb)
            def _():
                pltpu.async_copy(y_hbm.at[src_v.at[j]], rows_v, sem).wait()
                pltpu.sync_copy(rows_v, acc.at[dst_v.at[j]], add=True)
            return carry

        lax.fori_loop(0, n_chunks, body, 0)
        plsc.subcore_barrier()
        pltpu.sync_copy(acc.at[pl.ds(s * init_rows, init_rows)],
                        out_hbm.at[c, pl.ds(s * init_rows, init_rows)])

    return k


def _decode_gather_kernel(n, d, n_chunks):
    per_w = n_chunks * DCH

    @functools.partial(
        pl.kernel,
        mesh=_mesh(),
        out_type=(
            jax.ShapeDtypeStruct((NW * per_w, d), jnp.float32),
            jax.ShapeDtypeStruct((NW * per_w, d), jnp.float32),
        ),
        scratch_types=[
            pltpu.VMEM((n_chunks, DCH), jnp.int32),
            pltpu.VMEM((n_chunks, DCH), jnp.int32),
            pltpu.VMEM((DCH, d), jnp.float32),
            pltpu.VMEM((DCH, d), jnp.float32),
            pltpu.SemaphoreType.DMA,
            pltpu.SemaphoreType.DMA,
        ],
    )
    def k(z_hbm, i0_hbm, i1_hbm, a_hbm, b_hbm, i0_v, i1_v, ra_v, rb_v, s0, s1):
        c = lax.axis_index("c")
        s = lax.axis_index("s")
        w = c * NS + s
        base = w * per_w
        pltpu.sync_copy(i0_hbm.at[w], i0_v)
        pltpu.sync_copy(i1_hbm.at[w], i1_v)

        def body(j, carry):
            cp0 = pltpu.async_copy(z_hbm.at[i0_v.at[j]], ra_v, s0)
            cp1 = pltpu.async_copy(z_hbm.at[i1_v.at[j]], rb_v, s1)
            cp0.wait()
            cp1.wait()
            pltpu.sync_copy(ra_v, a_hbm.at[pl.ds(base + j * DCH, DCH)])
            pltpu.sync_copy(rb_v, b_hbm.at[pl.ds(base + j * DCH, DCH)])
            return carry

        lax.fori_loop(0, n_chunks, body, 0)

    return k


def _tc_dinv_body(degp_ref, dinv_ref):
    deg = jnp.sum(degp_ref[...], axis=0) + 1.0
    dinv_ref[...] = lax.rsqrt(deg)[:, None]


def _tc_y1_body(x_ref, w_ref, dinv_ref, y_ref):
    xw = jnp.dot(x_ref[...], w_ref[...], preferred_element_type=jnp.float32)
    y_ref[...] = xw * dinv_ref[...]


def _tc_mid_body(p_ref, y1_ref, dinv_ref, b1_ref, zs_ref):
    dinv = dinv_ref[...]
    h = (p_ref[0] + p_ref[1] + y1_ref[...]) * dinv + b1_ref[...]
    zs_ref[...] = jnp.maximum(h, 0.0) * dinv


def _tc_z2_body(p_ref, zs_ref, dinv_ref, b2_ref, w2_ref, z2_ref):
    u = (p_ref[0] + p_ref[1] + zs_ref[...]) * dinv_ref[...]
    z2 = jnp.dot(u, w2_ref[...], preferred_element_type=jnp.float32) + b2_ref[...]
    d_out = z2.shape[1]
    z2_ref[:, :d_out] = z2
    z2_ref[:, d_out:] = jnp.zeros_like(z2_ref[:, d_out:])


def _tc_dot_body(a_ref, b_ref, o_ref):
    o_ref[...] = jnp.sum(a_ref[...] * b_ref[...], axis=-1, keepdims=True)


def _row_blocks(n):
    for bs in (1000, 500, 250, 125, 100, 50, 25, 8, 5, 4, 2, 1):
        if n % bs == 0:
            return bs
    return n


def _split_chunks(a, fill, c0_chunks, c1_chunks, max_chunks, ch):
    n0 = NS * c0_chunks * ch
    n1 = NS * c1_chunks * ch
    a = a.astype(jnp.int32)
    pad_total = n0 + n1 - a.shape[0]
    a = jnp.concatenate([a, jnp.full((pad_total,), fill, dtype=jnp.int32)])

    def lay(seg, nchunks):
        seg = seg.reshape(NS, nchunks * ch)
        tail = jnp.full((NS, (max_chunks - nchunks) * ch), fill,
                        dtype=jnp.int32)
        return jnp.concatenate([seg, tail], axis=1)

    out = jnp.concatenate([lay(a[:n0], c0_chunks),
                           lay(a[n0:], c1_chunks)], axis=0)
    return out.reshape(NW, max_chunks, ch)


def _pad_chunks(a, fill, per_w_chunks, ch):
    total = NW * per_w_chunks * ch
    pad = total - a.shape[0]
    a = jnp.concatenate([a.astype(jnp.int32),
                         jnp.full((pad,), fill, dtype=jnp.int32)])
    return a.reshape(NW, per_w_chunks, ch)


def kernel(x, edge_index, edge_label_index, W1, b1, W2, b2):
    n, d_in = x.shape
    d_hid = W1.shape[1]
    d_out = W2.shape[1]
    e = edge_index.shape[1]
    e_lab = edge_label_index.shape[1]

    n_acc = ((n + 1 + NS * 8 - 1) // (NS * 8)) * (NS * 8)
    dump = n

    ect = -(-e // (NS * ECH))
    ec0 = max(1, (ect * 2) // 7)
    ec1 = ect - ec0
    if ec1 > 112:
        ec0 += ec1 - 112
        ec1 = 112
    ec = max(ec0, ec1)
    lc = -(-e_lab // (NW * DCH))

    src3 = _split_chunks(edge_index[0], 0, ec0, ec1, ec, ECH)
    dst3 = _split_chunks(edge_index[1], dump, ec0, ec1, ec, ECH)
    li0 = _pad_chunks(edge_label_index[0], 0, lc, DCH)
    li1 = _pad_chunks(edge_label_index[1], 0, lc, DCH)

    zeros_deg = jnp.zeros((n_acc,), jnp.float32)
    zeros1 = jnp.zeros((n_acc, d_hid), jnp.float32)

    deg_p = _deg_kernel(n_acc, ec * ECH)(dst3.reshape(NW, ec * ECH), zeros_deg)

    dinv = pl.pallas_call(
        _tc_dinv_body,
        grid=(1,),
        in_specs=[pl.BlockSpec((NW, n_acc), lambda i: (0, 0))],
        out_specs=pl.BlockSpec((n_acc, 1), lambda i: (0, 0)),
        out_shape=jax.ShapeDtypeStruct((n_acc, 1), jnp.float32),
    )(deg_p)

    bs = _row_blocks(n)
    grid = (n // bs,)
    deg_spec = pl.BlockSpec((bs, 1), lambda i: (i, 0))
    row_spec = lambda d: pl.BlockSpec((bs, d), lambda i: (i, 0))
    mat_spec = lambda a, b: pl.BlockSpec((a, b), lambda i: (0, 0))
    p_spec = lambda d: pl.BlockSpec((NC, bs, d), lambda i: (0, i, 0))

    y1 = pl.pallas_call(
        _tc_y1_body,
        grid=grid,
        in_specs=[row_spec(d_in), mat_spec(d_in, d_hid), deg_spec],
        out_specs=row_spec(d_hid),
        out_shape=jax.ShapeDtypeStruct((n, d_hid), jnp.float32),
    )(x, W1, dinv)

    p1 = _edge_kernel(n_acc, d_hid, ec, ec0, ec1)(y1, src3, dst3, zeros1)

    zs = pl.pallas_call(
        _tc_mid_body,
        grid=grid,
        in_specs=[p_spec(d_hid), row_spec(d_hid), deg_spec,
                  mat_spec(1, d_hid)],
        out_specs=row_spec(d_hid),
        out_shape=jax.ShapeDtypeStruct((n, d_hid), jnp.float32),
    )(p1, y1, dinv, b1.reshape(1, d_hid))

    p2 = _edge_kernel(n_acc, d_hid, ec, ec0, ec1)(zs, src3, dst3, zeros1)

    z2 = pl.pallas_call(
        _tc_z2_body,
        grid=grid,
        in_specs=[p_spec(d_hid), row_spec(d_hid), deg_spec,
                  mat_spec(1, d_out), mat_spec(d_hid, d_out)],
        out_specs=row_spec(d_hid),
        out_shape=jax.ShapeDtypeStruct((n, d_hid), jnp.float32),
    )(p2, zs, dinv, b2.reshape(1, d_out), W2)

    a_rows, b_rows = _decode_gather_kernel(n, d_hid, lc)(z2, li0, li1)

    e_pad = NW * lc * DCH
    dbs = min(2048, e_pad)
    while e_pad % dbs:
        dbs //= 2
    logits2 = pl.pallas_call(
        _tc_dot_body,
        grid=(e_pad // dbs,),
        in_specs=[pl.BlockSpec((dbs, d_hid), lambda i: (i, 0)),
                  pl.BlockSpec((dbs, d_hid), lambda i: (i, 0))],
        out_specs=pl.BlockSpec((dbs, 1), lambda i: (i, 0)),
        out_shape=jax.ShapeDtypeStruct((e_pad, 1), jnp.float32),
    )(a_rows, b_rows)

    return logits2[:e_lab, 0]
